# Initial kernel scaffold; baseline (speedup 1.0000x reference)
import functools
import math

import jax
import jax.numpy as jnp
from jax.experimental import pallas as pl

_SIZE = 100000
_PAD = 0
_SMOOTHING = 0.1
_CONF = 1.0 - _SMOOTHING
_EPS = _SMOOTHING / (_SIZE - 2)
_C = _CONF * math.log(_CONF) + (_SIZE - 2) * _EPS * math.log(_EPS)


def _loss_body(x_ref, t_ref, o_ref):
    x = x_ref[...]
    t = t_ref[0, 0, :]
    rowsum = jnp.sum(x, axis=1)
    col0 = x[:, 0]
    cols = jax.lax.broadcasted_iota(jnp.int32, x.shape, 1)
    g = jnp.sum(jnp.where(cols == t[:, None], x, 0.0), axis=1)
    per_row = jnp.where(
        t != _PAD,
        _C + _EPS * col0 - _EPS * rowsum - (_CONF - _EPS) * g,
        0.0,
    )
    partial = jnp.sum(per_row)

    @pl.when(pl.program_id(0) == 0)
    def _():
        o_ref[0, 0] = 0.0

    o_ref[0, 0] += partial


def kernel(x, target):
    n, v = x.shape
    block_rows = 16
    nb = n // block_rows
    t3 = target.astype(jnp.int32).reshape(nb, 1, block_rows)
    out = pl.pallas_call(
        _loss_body,
        grid=(nb,),
        in_specs=[
            pl.BlockSpec((block_rows, v), lambda i: (i, 0)),
            pl.BlockSpec((1, 1, block_rows), lambda i: (i, 0, 0)),
        ],
        out_specs=pl.BlockSpec((1, 1), lambda i: (0, 0)),
        out_shape=jax.ShapeDtypeStruct((1, 1), jnp.float32),
    )(x, t3)
    return out[0, 0]


# speedup vs baseline: 1.7414x; 1.7414x over previous
import functools
import math

import jax
import jax.numpy as jnp
from jax.experimental import pallas as pl
from jax.experimental.pallas import tpu as pltpu

_SIZE = 100000
_PAD = 0
_SMOOTHING = 0.1
_CONF = 1.0 - _SMOOTHING
_EPS = _SMOOTHING / (_SIZE - 2)
_C = _CONF * math.log(_CONF) + (_SIZE - 2) * _EPS * math.log(_EPS)


def _loss_body(x_ref, t_ref, o_ref):
    x = x_ref[...]
    t = t_ref[0, 0, :]
    rowsum = jnp.sum(x, axis=1)
    col0 = x[:, 0]
    cols = jax.lax.broadcasted_iota(jnp.int32, x.shape, 1)
    g = jnp.sum(jnp.where(cols == t[:, None], x, 0.0), axis=1)
    per_row = jnp.where(
        t != _PAD,
        _C + _EPS * col0 - _EPS * rowsum - (_CONF - _EPS) * g,
        0.0,
    )
    partial = jnp.sum(per_row)

    @pl.when(pl.program_id(0) == 0)
    def _():
        o_ref[0, 0] = 0.0

    o_ref[0, 0] += partial


def kernel(x, target):
    n, v = x.shape
    block_rows = 16
    nb = n // block_rows
    t3 = target.astype(jnp.int32).reshape(nb, 1, block_rows)
    out = pl.pallas_call(
        _loss_body,
        grid=(nb,),
        in_specs=[
            pl.BlockSpec((block_rows, v), lambda i: (i, 0)),
            pl.BlockSpec((1, 1, block_rows), lambda i: (i, 0, 0)),
        ],
        out_specs=pl.BlockSpec(memory_space=pltpu.SMEM),
        out_shape=jax.ShapeDtypeStruct((1, 1), jnp.float32),
    )(x, t3)
    return out[0, 0]
